# baseline (speedup 1.0000x reference)
import jax
import jax.numpy as jnp
from jax.experimental import pallas as pl
from jax.experimental.pallas import tpu as pltpu

TOP_K = 8


def _router_body(n_steps, x_ref, w1_ref, w2_ref, w_out_ref, e_out_ref,
                 logits_ref, h_prev_ref):
    n = pl.program_id(1)

    @pl.when(n > 0)
    def _():
        partial = jnp.dot(h_prev_ref[...], w2_ref[...],
                          preferred_element_type=jnp.float32)

        @pl.when(n == 1)
        def _():
            logits_ref[...] = partial

        @pl.when(n > 1)
        def _():
            logits_ref[...] += partial

    @pl.when(n < n_steps)
    def _():
        h = jnp.dot(x_ref[...], w1_ref[...],
                    preferred_element_type=jnp.float32)
        h = h * jax.nn.sigmoid(h)
        h_prev_ref[...] = h.astype(jnp.bfloat16)

    @pl.when(n == n_steps)
    def _():
        logits = logits_ref[...]
        num_e = logits.shape[-1]
        m = jnp.max(logits, axis=-1, keepdims=True)
        ex = jnp.exp(logits - m)
        probs = ex / jnp.sum(ex, axis=-1, keepdims=True)
        ids = jax.lax.broadcasted_iota(jnp.int32, probs.shape, 1)
        p = probs
        ws, es = [], []
        for _ in range(TOP_K):
            mx = jnp.max(p, axis=-1, keepdims=True)
            idx = jnp.min(jnp.where(p == mx, ids, num_e), axis=-1,
                          keepdims=True)
            ws.append(mx)
            es.append(idx)
            p = jnp.where(ids == idx, -1.0, p)
        w_out_ref[...] = jnp.concatenate(ws, axis=-1)
        e_out_ref[...] = jnp.concatenate(es, axis=-1)


def _router_single(x, W1, W2):
    tokens, hidden = x.shape
    num_e = W2.shape[1]
    t_tile = min(1024, tokens)
    n_tile = min(512, hidden)
    n_steps = hidden // n_tile
    grid = (tokens // t_tile, n_steps + 1)

    W2b = W2.astype(jnp.bfloat16)
    body = lambda *refs: _router_body(n_steps, *refs)
    clamp = lambda i: jnp.minimum(i, n_steps - 1)
    weights, experts, logits = pl.pallas_call(
        body,
        grid=grid,
        in_specs=[
            pl.BlockSpec((t_tile, hidden), lambda t, n: (t, 0)),
            pl.BlockSpec((hidden, n_tile), lambda t, n: (0, clamp(n))),
            pl.BlockSpec((n_tile, num_e),
                         lambda t, n: (clamp(jnp.maximum(n - 1, 0)), 0)),
        ],
        out_specs=[
            pl.BlockSpec((t_tile, TOP_K), lambda t, n: (t, 0)),
            pl.BlockSpec((t_tile, TOP_K), lambda t, n: (t, 0)),
            pl.BlockSpec((t_tile, num_e), lambda t, n: (t, 0)),
        ],
        out_shape=[
            jax.ShapeDtypeStruct((tokens, TOP_K), jnp.float32),
            jax.ShapeDtypeStruct((tokens, TOP_K), jnp.int32),
            jax.ShapeDtypeStruct((tokens, num_e), jnp.float32),
        ],
        scratch_shapes=[pltpu.VMEM((t_tile, n_tile), jnp.bfloat16)],
        compiler_params=pltpu.CompilerParams(
            dimension_semantics=("parallel", "arbitrary")),
    )(x, W1, W2b)
    return (weights, experts, logits)


def kernel(x, W1, W2):
    return _router_single(x, W1, W2)


# speedup vs baseline: 1.1637x; 1.1637x over previous
import jax
import jax.numpy as jnp
from jax.experimental import pallas as pl
from jax.experimental.pallas import tpu as pltpu

TOP_K = 8


def _router_body(n_steps, x_ref, w1_ref, w2_ref, w_out_ref, e_out_ref,
                 logits_ref):
    n = pl.program_id(1)
    h = jnp.dot(x_ref[...], w1_ref[...], preferred_element_type=jnp.float32)
    h = h * jax.nn.sigmoid(h)
    partial = jnp.dot(h.astype(jnp.bfloat16), w2_ref[...],
                      preferred_element_type=jnp.float32)

    @pl.when(n == 0)
    def _():
        logits_ref[...] = partial

    @pl.when(n > 0)
    def _():
        logits_ref[...] += partial

    @pl.when(n == n_steps - 1)
    def _():
        logits = logits_ref[...]
        num_e = logits.shape[-1]
        m = jnp.max(logits, axis=-1, keepdims=True)
        ex = jnp.exp(logits - m)
        probs = ex / jnp.sum(ex, axis=-1, keepdims=True)
        ids = jax.lax.broadcasted_iota(jnp.int32, probs.shape, 1)
        p = probs
        ws, es = [], []
        for _ in range(TOP_K):
            mx = jnp.max(p, axis=-1, keepdims=True)
            idx = jnp.min(jnp.where(p == mx, ids, num_e), axis=-1,
                          keepdims=True)
            ws.append(mx)
            es.append(idx)
            p = jnp.where(ids == idx, -1.0, p)
        w_out_ref[...] = jnp.concatenate(ws, axis=-1)
        e_out_ref[...] = jnp.concatenate(es, axis=-1)


def _router_single(x, W1, W2):
    tokens, hidden = x.shape
    num_e = W2.shape[1]
    t_tile = min(1024, tokens)
    n_tile = min(512, hidden)
    n_steps = hidden // n_tile
    grid = (tokens // t_tile, n_steps)

    body = lambda *refs: _router_body(n_steps, *refs)
    weights, experts, logits = pl.pallas_call(
        body,
        grid=grid,
        in_specs=[
            pl.BlockSpec((t_tile, hidden), lambda t, n: (t, 0)),
            pl.BlockSpec((hidden, n_tile), lambda t, n: (0, n)),
            pl.BlockSpec((n_tile, num_e), lambda t, n: (n, 0)),
        ],
        out_specs=[
            pl.BlockSpec((t_tile, TOP_K), lambda t, n: (t, 0)),
            pl.BlockSpec((t_tile, TOP_K), lambda t, n: (t, 0)),
            pl.BlockSpec((t_tile, num_e), lambda t, n: (t, 0)),
        ],
        out_shape=[
            jax.ShapeDtypeStruct((tokens, TOP_K), jnp.float32),
            jax.ShapeDtypeStruct((tokens, TOP_K), jnp.int32),
            jax.ShapeDtypeStruct((tokens, num_e), jnp.float32),
        ],
        compiler_params=pltpu.CompilerParams(
            dimension_semantics=("parallel", "arbitrary")),
    )(x, W1, W2)
    return (weights, experts, logits)


def kernel(x, W1, W2):
    return _router_single(x, W1, W2)


# speedup vs baseline: 1.1695x; 1.0050x over previous
import jax
import jax.numpy as jnp
from jax.experimental import pallas as pl
from jax.experimental.pallas import tpu as pltpu

TOP_K = 8


def _router_body(n_steps, x_ref, w1_ref, w2_ref, w_out_ref, e_out_ref,
                 logits_ref):
    n = pl.program_id(1)
    h = jnp.dot(x_ref[...], w1_ref[...], preferred_element_type=jnp.float32)
    h = h * jax.nn.sigmoid(h)
    partial = jnp.dot(h, w2_ref[...], preferred_element_type=jnp.float32)

    @pl.when(n == 0)
    def _():
        logits_ref[...] = partial

    @pl.when(n > 0)
    def _():
        logits_ref[...] += partial

    @pl.when(n == n_steps - 1)
    def _():
        logits = logits_ref[...]
        num_e = logits.shape[-1]
        m = jnp.max(logits, axis=-1, keepdims=True)
        ex = jnp.exp(logits - m)
        probs = ex / jnp.sum(ex, axis=-1, keepdims=True)
        ids = jax.lax.broadcasted_iota(jnp.int32, probs.shape, 1)
        p = probs
        ws, es = [], []
        for _ in range(TOP_K):
            mx = jnp.max(p, axis=-1, keepdims=True)
            idx = jnp.min(jnp.where(p == mx, ids, num_e), axis=-1,
                          keepdims=True)
            ws.append(mx)
            es.append(idx)
            p = jnp.where(ids == idx, -1.0, p)
        w_out_ref[...] = jnp.concatenate(ws, axis=-1)
        e_out_ref[...] = jnp.concatenate(es, axis=-1)


def _router_single(x, W1, W2):
    tokens, hidden = x.shape
    num_e = W2.shape[1]
    t_tile = min(1024, tokens)
    n_tile = min(512, hidden)
    n_steps = hidden // n_tile
    grid = (tokens // t_tile, n_steps)

    body = lambda *refs: _router_body(n_steps, *refs)
    weights, experts, logits = pl.pallas_call(
        body,
        grid=grid,
        in_specs=[
            pl.BlockSpec((t_tile, hidden), lambda t, n: (t, 0)),
            pl.BlockSpec((hidden, n_tile), lambda t, n: (0, n)),
            pl.BlockSpec((n_tile, num_e), lambda t, n: (n, 0)),
        ],
        out_specs=[
            pl.BlockSpec((t_tile, TOP_K), lambda t, n: (t, 0)),
            pl.BlockSpec((t_tile, TOP_K), lambda t, n: (t, 0)),
            pl.BlockSpec((t_tile, num_e), lambda t, n: (t, 0)),
        ],
        out_shape=[
            jax.ShapeDtypeStruct((tokens, TOP_K), jnp.float32),
            jax.ShapeDtypeStruct((tokens, TOP_K), jnp.int32),
            jax.ShapeDtypeStruct((tokens, num_e), jnp.float32),
        ],
        compiler_params=pltpu.CompilerParams(
            dimension_semantics=("parallel", "arbitrary")),
    )(x, W1, W2)
    return (weights, experts, logits)


def kernel(x, W1, W2):
    return _router_single(x, W1, W2)
